# baseline (speedup 1.0000x reference)
import functools

import jax
import jax.numpy as jnp
from jax import lax
from jax.experimental import pallas as pl
from jax.experimental.pallas import tpu as pltpu
from jax.experimental.pallas import tpu_sc as plsc

N = 10000
E = 320000
D_IN = 128
H = 32
OUT = 10
G = 64

NC = 2
NS = 16
NW = NC * NS
CH = 128
NCHUNK = 2 * (-(-E // (NW * CH * 2)))
EPW = NCHUNK * CH
E_PAD = EPW * NW
NPAD = 10240
RPS = NPAD // NS
HIST_W = 16

_mesh = plsc.VectorSubcoreMesh(core_axis_name="c", subcore_axis_name="s")
_sc_params = pltpu.CompilerParams(use_tc_tiling_on_sc=False)


@jax.jit
def _sc_hist(dst3):
    @functools.partial(
        pl.kernel,
        out_type=jax.ShapeDtypeStruct((NC, NPAD, HIST_W), jnp.float32),
        mesh=_mesh,
        scratch_types=[
            pltpu.VMEM((NCHUNK, CH), jnp.int32),
            pltpu.VMEM((CH, HIST_W), jnp.float32),
            pltpu.VMEM((CH, HIST_W), jnp.float32),
            pltpu.VMEM_SHARED((NPAD, HIST_W), jnp.float32),
        ],
        compiler_params=_sc_params,
    )
    def k(dst_hbm, out_hbm, dst_v, ones_v, zeros_v, acc_sh):
        c = lax.axis_index("c")
        s = lax.axis_index("s")
        wid = s * NC + c

        one16 = jnp.ones((16,), jnp.float32)
        zero16 = jnp.zeros((16,), jnp.float32)

        @pl.loop(0, CH)
        def _(i):
            ones_v[i, pl.ds(0, 16)] = one16
            zeros_v[i, pl.ds(0, 16)] = zero16

        @pl.loop(0, RPS // CH)
        def _(t):
            pltpu.sync_copy(zeros_v, acc_sh.at[pl.ds(s * RPS + t * CH, CH)])

        plsc.subcore_barrier()

        pltpu.sync_copy(dst_hbm.at[wid], dst_v)

        @pl.loop(0, NCHUNK)
        def _(j):
            pltpu.sync_copy(ones_v, acc_sh.at[dst_v.at[j]], add=True)

        plsc.subcore_barrier()
        pltpu.sync_copy(acc_sh.at[pl.ds(s * RPS, RPS)],
                        out_hbm.at[c, pl.ds(s * RPS, RPS)])

    return k(dst3)


@jax.jit
def _sc_agg(g, src3, dst3):
    @functools.partial(
        pl.kernel,
        out_type=jax.ShapeDtypeStruct((NC, NPAD, H), jnp.float32),
        mesh=_mesh,
        scratch_types=[
            pltpu.VMEM((NCHUNK, CH), jnp.int32),
            pltpu.VMEM((NCHUNK, CH), jnp.int32),
            [pltpu.VMEM((CH, H), jnp.float32) for _ in range(4)],
            [pltpu.SemaphoreType.DMA for _ in range(4)],
            [pltpu.SemaphoreType.DMA for _ in range(4)],
            pltpu.VMEM_SHARED((NPAD, H), jnp.float32),
            pltpu.VMEM_SHARED((NPAD, H), jnp.float32),
        ],
        compiler_params=_sc_params,
    )
    def k(g_hbm, src_hbm, dst_hbm, out_hbm, src_v, dst_v, bufs, gsems, ssems,
          acc_sh, g_sh):
        c = lax.axis_index("c")
        s = lax.axis_index("s")
        wid = s * NC + c
        NB = 4

        zero16 = jnp.zeros((16,), jnp.float32)

        @pl.loop(0, CH)
        def _(i):
            bufs[0][i, pl.ds(0, 16)] = zero16
            bufs[0][i, pl.ds(16, 16)] = zero16

        @pl.loop(0, RPS // CH)
        def _(t):
            pltpu.sync_copy(bufs[0], acc_sh.at[pl.ds(s * RPS + t * CH, CH)])

        nst = N // NS
        pltpu.sync_copy(g_hbm.at[pl.ds(s * nst, nst)],
                        g_sh.at[pl.ds(s * nst, nst)])

        plsc.subcore_barrier()

        pltpu.sync_copy(src_hbm.at[wid], src_v)
        pltpu.sync_copy(dst_hbm.at[wid], dst_v)

        for i in range(NB):
            pltpu.async_copy(g_sh.at[src_v.at[i]], bufs[i], gsems[i])

        @pl.loop(0, NCHUNK - NB, step=NB)
        def _(j):
            for i in range(NB):
                pltpu.make_async_copy(g_sh.at[src_v.at[j + i]],
                                      bufs[i], gsems[i]).wait()
                pltpu.async_copy(bufs[i], acc_sh.at[dst_v.at[j + i]],
                                 ssems[i], add=True)
            for i in range(NB):
                pltpu.make_async_copy(bufs[i], acc_sh.at[dst_v.at[j + i]],
                                      ssems[i]).wait()
                pltpu.async_copy(g_sh.at[src_v.at[j + NB + i]],
                                 bufs[i], gsems[i])

        for i in range(NB):
            jt = NCHUNK - NB + i
            pltpu.make_async_copy(g_sh.at[src_v.at[jt]], bufs[i],
                                  gsems[i]).wait()
            pltpu.async_copy(bufs[i], acc_sh.at[dst_v.at[jt]],
                             ssems[i], add=True)
        for i in range(NB):
            jt = NCHUNK - NB + i
            pltpu.make_async_copy(bufs[i], acc_sh.at[dst_v.at[jt]],
                                  ssems[i]).wait()

        plsc.subcore_barrier()
        pltpu.sync_copy(acc_sh.at[pl.ds(s * RPS, RPS)],
                        out_hbm.at[c, pl.ds(s * RPS, RPS)])

    return k(g, src3, dst3)


def _tc_xform_body(x_ref, w_ref, o_ref):
    o_ref[...] = jnp.dot(x_ref[...], w_ref[...],
                         preferred_element_type=jnp.float32)


@jax.jit
def _tc_xform(x, w):
    return pl.pallas_call(
        _tc_xform_body,
        out_shape=jax.ShapeDtypeStruct((N, H), jnp.float32),
    )(x, w)


def _tc_scale_body(hist_ref, hp_ref, dis_ref, g_ref):
    deg = hist_ref[0, :N, :] + hist_ref[1, :N, :] + 1.0
    dis16 = lax.rsqrt(deg)
    dis = jnp.concatenate([dis16, dis16], axis=1)
    dis_ref[...] = dis
    g_ref[...] = dis * hp_ref[...]


@jax.jit
def _tc_scale(hist, h1p):
    return pl.pallas_call(
        _tc_scale_body,
        out_shape=[
            jax.ShapeDtypeStruct((N, H), jnp.float32),
            jax.ShapeDtypeStruct((N, H), jnp.float32),
        ],
    )(hist, h1p)


def _tc_layer_body(s_ref, hp_ref, dis_ref, b_ref, w_ref, hp2_ref, g2_ref):
    dis = dis_ref[...]
    ssum = s_ref[0, :N, :] + s_ref[1, :N, :]
    out = dis * ssum + dis * dis * hp_ref[...] + b_ref[...]
    h = jnp.maximum(out, 0.0)
    hp2 = jnp.dot(h, w_ref[...], preferred_element_type=jnp.float32)
    hp2_ref[...] = hp2
    g2_ref[...] = dis * hp2


@jax.jit
def _tc_layer(s, hp, dis, b, w):
    return pl.pallas_call(
        _tc_layer_body,
        out_shape=[
            jax.ShapeDtypeStruct((N, H), jnp.float32),
            jax.ShapeDtypeStruct((N, H), jnp.float32),
        ],
    )(s, hp, dis, b, w)


def _tc_head_body(s_ref, hp_ref, dis_ref, b_ref, batch_ref, wl_ref, bl_ref,
                  o_ref):
    dis = dis_ref[...]
    ssum = s_ref[0, :N, :] + s_ref[1, :N, :]
    out3 = dis * ssum + dis * dis * hp_ref[...] + b_ref[...]
    gids = lax.broadcasted_iota(jnp.int32, (N, G), 1)
    onehot = (batch_ref[...] == gids).astype(jnp.float32)
    sums = lax.dot_general(onehot, out3, (((0,), (0,)), ((), ())),
                           preferred_element_type=jnp.float32)
    counts = jnp.sum(onehot, axis=0)[:, None]
    emb = sums / jnp.maximum(counts, 1.0)
    o_ref[...] = jnp.dot(emb, wl_ref[...],
                         preferred_element_type=jnp.float32) + bl_ref[...]


@jax.jit
def _tc_head(s, hp, dis, b, batch2d, wl, bl):
    return pl.pallas_call(
        _tc_head_body,
        out_shape=jax.ShapeDtypeStruct((G, OUT), jnp.float32),
    )(s, hp, dis, b, batch2d, wl, bl)


def kernel(x, edge_index, batch, W1, b1, W2, b2, W3, b3, Wl, bl):
    x = x.astype(jnp.float32)
    src = edge_index[0]
    dst = edge_index[1]
    pad = E_PAD - E
    src3 = jnp.concatenate(
        [src, jnp.zeros((pad,), src.dtype)]).reshape(NW, NCHUNK, CH)
    dst3 = jnp.concatenate(
        [dst, jnp.full((pad,), N, dst.dtype)]).reshape(NW, NCHUNK, CH)

    hist = _sc_hist(dst3)
    h1p = _tc_xform(x, W1)
    dis, g1 = _tc_scale(hist, h1p)

    s1 = _sc_agg(g1, src3, dst3)
    h2p, g2 = _tc_layer(s1, h1p, dis, b1.reshape(1, H), W2)
    s2 = _sc_agg(g2, src3, dst3)
    h3p, g3 = _tc_layer(s2, h2p, dis, b2.reshape(1, H), W3)
    s3 = _sc_agg(g3, src3, dst3)

    return _tc_head(s3, h3p, dis, b3.reshape(1, H),
                    batch.reshape(N, 1).astype(jnp.int32),
                    Wl, bl.reshape(1, OUT))


# speedup vs baseline: 1.0828x; 1.0828x over previous
import functools

import jax
import jax.numpy as jnp
from jax import lax
from jax.experimental import pallas as pl
from jax.experimental.pallas import tpu as pltpu
from jax.experimental.pallas import tpu_sc as plsc

N = 10000
E = 320000
D_IN = 128
H = 32
OUT = 10
G = 64

NC = 2
NS = 16
NW = NC * NS
CH = 128
KR = 2
CHE = KR * CH
NCHUNK = 2 * (-(-E // (NW * CHE * 2)))
EPW = NCHUNK * CHE
E_PAD = EPW * NW
NPAD = 10240
RPS = NPAD // NS
HIST_W = 16

_mesh = plsc.VectorSubcoreMesh(core_axis_name="c", subcore_axis_name="s")
_sc_params = pltpu.CompilerParams(use_tc_tiling_on_sc=False)


@jax.jit
def _sc_hist(dst3):
    @functools.partial(
        pl.kernel,
        out_type=jax.ShapeDtypeStruct((NC, NPAD, HIST_W), jnp.float32),
        mesh=_mesh,
        scratch_types=[
            pltpu.VMEM((NCHUNK, CHE), jnp.int32),
            pltpu.VMEM((CHE, HIST_W), jnp.float32),
            pltpu.VMEM((CH, HIST_W), jnp.float32),
            pltpu.VMEM_SHARED((NPAD, HIST_W), jnp.float32),
        ],
        compiler_params=_sc_params,
    )
    def k(dst_hbm, out_hbm, dst_v, ones_v, zeros_v, acc_sh):
        c = lax.axis_index("c")
        s = lax.axis_index("s")
        wid = s * NC + c

        one16 = jnp.ones((16,), jnp.float32)
        zero16 = jnp.zeros((16,), jnp.float32)

        @pl.loop(0, CHE)
        def _(i):
            ones_v[i, pl.ds(0, 16)] = one16

        @pl.loop(0, CH)
        def _(i):
            zeros_v[i, pl.ds(0, 16)] = zero16

        @pl.loop(0, RPS // CH)
        def _(t):
            pltpu.sync_copy(zeros_v, acc_sh.at[pl.ds(s * RPS + t * CH, CH)])

        plsc.subcore_barrier()

        pltpu.sync_copy(dst_hbm.at[wid], dst_v)

        @pl.loop(0, NCHUNK)
        def _(j):
            pltpu.sync_copy(ones_v, acc_sh.at[dst_v.at[j]], add=True)

        plsc.subcore_barrier()
        pltpu.sync_copy(acc_sh.at[pl.ds(s * RPS, RPS)],
                        out_hbm.at[c, pl.ds(s * RPS, RPS)])

    return k(dst3)


@jax.jit
def _sc_agg(g, src3, dst3):
    @functools.partial(
        pl.kernel,
        out_type=jax.ShapeDtypeStruct((NC, NPAD, H), jnp.float32),
        mesh=_mesh,
        scratch_types=[
            pltpu.VMEM((NCHUNK, CHE), jnp.int32),
            pltpu.VMEM((NCHUNK, CHE), jnp.int32),
            pltpu.VMEM((CHE, H), jnp.float32),
            pltpu.VMEM((CHE, H), jnp.float32),
            pltpu.SemaphoreType.DMA,
            pltpu.SemaphoreType.DMA,
            pltpu.VMEM_SHARED((NPAD, H), jnp.float32),
            pltpu.VMEM_SHARED((NPAD, H), jnp.float32),
        ],
        compiler_params=_sc_params,
    )
    def k(g_hbm, src_hbm, dst_hbm, out_hbm, src_v, dst_v, rows0, rows1,
          sem0, sem1, acc_sh, g_sh):
        c = lax.axis_index("c")
        s = lax.axis_index("s")
        wid = s * NC + c

        zero16 = jnp.zeros((16,), jnp.float32)

        @pl.loop(0, CH)
        def _(i):
            rows0[i, pl.ds(0, 16)] = zero16
            rows0[i, pl.ds(16, 16)] = zero16

        @pl.loop(0, RPS // CH)
        def _(t):
            pltpu.sync_copy(rows0.at[pl.ds(0, CH)],
                            acc_sh.at[pl.ds(s * RPS + t * CH, CH)])

        nst = N // NS
        pltpu.sync_copy(g_hbm.at[pl.ds(s * nst, nst)],
                        g_sh.at[pl.ds(s * nst, nst)])

        plsc.subcore_barrier()

        pltpu.sync_copy(src_hbm.at[wid], src_v)
        pltpu.sync_copy(dst_hbm.at[wid], dst_v)

        pltpu.async_copy(g_sh.at[src_v.at[0]], rows0, sem0)
        pltpu.async_copy(g_sh.at[src_v.at[1]], rows1, sem1)

        @pl.loop(0, NCHUNK - 2, step=2)
        def _(j):
            pltpu.make_async_copy(g_sh.at[src_v.at[j]], rows0, sem0).wait()
            pltpu.sync_copy(rows0, acc_sh.at[dst_v.at[j]], add=True)
            pltpu.async_copy(g_sh.at[src_v.at[j + 2]], rows0, sem0)
            pltpu.make_async_copy(g_sh.at[src_v.at[j + 1]], rows1, sem1).wait()
            pltpu.sync_copy(rows1, acc_sh.at[dst_v.at[j + 1]], add=True)
            pltpu.async_copy(g_sh.at[src_v.at[j + 3]], rows1, sem1)

        pltpu.make_async_copy(g_sh.at[src_v.at[NCHUNK - 2]], rows0, sem0).wait()
        pltpu.sync_copy(rows0, acc_sh.at[dst_v.at[NCHUNK - 2]], add=True)
        pltpu.make_async_copy(g_sh.at[src_v.at[NCHUNK - 1]], rows1, sem1).wait()
        pltpu.sync_copy(rows1, acc_sh.at[dst_v.at[NCHUNK - 1]], add=True)

        plsc.subcore_barrier()
        pltpu.sync_copy(acc_sh.at[pl.ds(s * RPS, RPS)],
                        out_hbm.at[c, pl.ds(s * RPS, RPS)])

    return k(g, src3, dst3)


def _tc_xform_body(x_ref, w_ref, o_ref):
    o_ref[...] = jnp.dot(x_ref[...], w_ref[...],
                         preferred_element_type=jnp.float32)


@jax.jit
def _tc_xform(x, w):
    return pl.pallas_call(
        _tc_xform_body,
        out_shape=jax.ShapeDtypeStruct((N, H), jnp.float32),
    )(x, w)


def _tc_scale_body(hist_ref, hp_ref, dis_ref, g_ref):
    deg = hist_ref[0, :N, :] + hist_ref[1, :N, :] + 1.0
    dis16 = lax.rsqrt(deg)
    dis = jnp.concatenate([dis16, dis16], axis=1)
    dis_ref[...] = dis
    g_ref[...] = dis * hp_ref[...]


@jax.jit
def _tc_scale(hist, h1p):
    return pl.pallas_call(
        _tc_scale_body,
        out_shape=[
            jax.ShapeDtypeStruct((N, H), jnp.float32),
            jax.ShapeDtypeStruct((N, H), jnp.float32),
        ],
    )(hist, h1p)


def _tc_layer_body(s_ref, hp_ref, dis_ref, b_ref, w_ref, hp2_ref, g2_ref):
    dis = dis_ref[...]
    ssum = s_ref[0, :N, :] + s_ref[1, :N, :]
    out = dis * ssum + dis * dis * hp_ref[...] + b_ref[...]
    h = jnp.maximum(out, 0.0)
    hp2 = jnp.dot(h, w_ref[...], preferred_element_type=jnp.float32)
    hp2_ref[...] = hp2
    g2_ref[...] = dis * hp2


@jax.jit
def _tc_layer(s, hp, dis, b, w):
    return pl.pallas_call(
        _tc_layer_body,
        out_shape=[
            jax.ShapeDtypeStruct((N, H), jnp.float32),
            jax.ShapeDtypeStruct((N, H), jnp.float32),
        ],
    )(s, hp, dis, b, w)


def _tc_head_body(s_ref, hp_ref, dis_ref, b_ref, batch_ref, wl_ref, bl_ref,
                  o_ref):
    dis = dis_ref[...]
    ssum = s_ref[0, :N, :] + s_ref[1, :N, :]
    out3 = dis * ssum + dis * dis * hp_ref[...] + b_ref[...]
    gids = lax.broadcasted_iota(jnp.int32, (N, G), 1)
    onehot = (batch_ref[...] == gids).astype(jnp.float32)
    sums = lax.dot_general(onehot, out3, (((0,), (0,)), ((), ())),
                           preferred_element_type=jnp.float32)
    counts = jnp.sum(onehot, axis=0)[:, None]
    emb = sums / jnp.maximum(counts, 1.0)
    o_ref[...] = jnp.dot(emb, wl_ref[...],
                         preferred_element_type=jnp.float32) + bl_ref[...]


@jax.jit
def _tc_head(s, hp, dis, b, batch2d, wl, bl):
    return pl.pallas_call(
        _tc_head_body,
        out_shape=jax.ShapeDtypeStruct((G, OUT), jnp.float32),
    )(s, hp, dis, b, batch2d, wl, bl)


def kernel(x, edge_index, batch, W1, b1, W2, b2, W3, b3, Wl, bl):
    x = x.astype(jnp.float32)
    src = edge_index[0]
    dst = edge_index[1]
    pad = E_PAD - E
    src3 = jnp.concatenate(
        [src, jnp.zeros((pad,), src.dtype)]).reshape(NW, NCHUNK, CHE)
    dst3 = jnp.concatenate(
        [dst, jnp.full((pad,), N, dst.dtype)]).reshape(NW, NCHUNK, CHE)

    hist = _sc_hist(dst3)
    h1p = _tc_xform(x, W1)
    dis, g1 = _tc_scale(hist, h1p)

    s1 = _sc_agg(g1, src3, dst3)
    h2p, g2 = _tc_layer(s1, h1p, dis, b1.reshape(1, H), W2)
    s2 = _sc_agg(g2, src3, dst3)
    h3p, g3 = _tc_layer(s2, h2p, dis, b2.reshape(1, H), W3)
    s3 = _sc_agg(g3, src3, dst3)

    return _tc_head(s3, h3p, dis, b3.reshape(1, H),
                    batch.reshape(N, 1).astype(jnp.int32),
                    Wl, bl.reshape(1, OUT))


# speedup vs baseline: 1.2861x; 1.1878x over previous
import functools

import jax
import jax.numpy as jnp
from jax import lax
from jax.experimental import pallas as pl
from jax.experimental.pallas import tpu as pltpu
from jax.experimental.pallas import tpu_sc as plsc

N = 10000
E = 320000
D_IN = 128
H = 32
OUT = 10
G = 64

NC = 2
NS = 16
NW = NC * NS
CH = 128
CHE = 256
NCHUNK = 2 * (-(-E // (NW * CHE * 2)))
EPW = NCHUNK * CHE
E_PAD = EPW * NW
PK = 4
NP = N // PK
NPAD = 10240
NPP = NPAD // PK
RPS = NPAD // NS
RPP = NPP // NS

_mesh = plsc.VectorSubcoreMesh(core_axis_name="c", subcore_axis_name="s")
_sc_params = pltpu.CompilerParams(use_tc_tiling_on_sc=False)


@jax.jit
def _sc_hist(dst3):
    @functools.partial(
        pl.kernel,
        out_type=jax.ShapeDtypeStruct((NC, NPAD, H), jnp.float32),
        mesh=_mesh,
        scratch_types=[
            pltpu.VMEM((NCHUNK, CHE), jnp.int32),
            pltpu.VMEM((CHE, H), jnp.float32),
            pltpu.VMEM((CH, H), jnp.float32),
            pltpu.VMEM_SHARED((NPAD, H), jnp.float32),
        ],
        compiler_params=_sc_params,
    )
    def k(dst_hbm, out_hbm, dst_v, ones_v, zeros_v, acc_sh):
        c = lax.axis_index("c")
        s = lax.axis_index("s")
        wid = s * NC + c
        acc_rows = acc_sh
        out_rows = out_hbm

        one16 = jnp.ones((16,), jnp.float32)
        zero16 = jnp.zeros((16,), jnp.float32)

        @pl.loop(0, CHE)
        def _(i):
            ones_v[i, pl.ds(0, 16)] = one16
            ones_v[i, pl.ds(16, 16)] = one16

        @pl.loop(0, CH)
        def _(i):
            zeros_v[i, pl.ds(0, 16)] = zero16
            zeros_v[i, pl.ds(16, 16)] = zero16

        @pl.loop(0, RPS // CH)
        def _(t):
            pltpu.sync_copy(zeros_v, acc_rows.at[pl.ds(s * RPS + t * CH, CH)])

        plsc.subcore_barrier()

        pltpu.sync_copy(dst_hbm.at[wid], dst_v)

        @pl.loop(0, NCHUNK)
        def _(j):
            pltpu.sync_copy(ones_v, acc_rows.at[dst_v.at[j]], add=True)

        plsc.subcore_barrier()
        pltpu.sync_copy(acc_rows.at[pl.ds(s * RPS, RPS)],
                        out_rows.at[c, pl.ds(s * RPS, RPS)])

    return k(dst3)


@jax.jit
def _sc_agg(g, src3, dst3):
    @functools.partial(
        pl.kernel,
        out_type=jax.ShapeDtypeStruct((NC, NPAD, H), jnp.float32),
        mesh=_mesh,
        scratch_types=[
            pltpu.VMEM((NCHUNK, CHE), jnp.int32),
            pltpu.VMEM((NCHUNK, CHE), jnp.int32),
            pltpu.VMEM((CHE, H), jnp.float32),
            pltpu.VMEM((CHE, H), jnp.float32),
            pltpu.SemaphoreType.DMA,
            pltpu.SemaphoreType.DMA,
            pltpu.VMEM_SHARED((NPAD, H), jnp.float32),
            pltpu.VMEM_SHARED((NPAD, H), jnp.float32),
        ],
        compiler_params=_sc_params,
    )
    def k(g_hbm, src_hbm, dst_hbm, out_hbm, src_v, dst_v, rows0, rows1,
          sem0, sem1, acc_sh, g_sh):
        c = lax.axis_index("c")
        s = lax.axis_index("s")
        wid = s * NC + c
        acc_rows = acc_sh
        g_rows = g_sh
        g_hbm_rows = g_hbm
        out_rows = out_hbm

        zero16 = jnp.zeros((16,), jnp.float32)

        @pl.loop(0, CH)
        def _(i):
            rows0[i, pl.ds(0, 16)] = zero16
            rows0[i, pl.ds(16, 16)] = zero16

        @pl.loop(0, RPS // CH)
        def _(t):
            pltpu.sync_copy(rows0.at[pl.ds(0, CH)],
                            acc_rows.at[pl.ds(s * RPS + t * CH, CH)])

        pltpu.sync_copy(g_hbm_rows.at[pl.ds(s * RPS, RPS)],
                        g_sh.at[pl.ds(s * RPS, RPS)])

        plsc.subcore_barrier()

        pltpu.sync_copy(src_hbm.at[wid], src_v)
        pltpu.sync_copy(dst_hbm.at[wid], dst_v)

        pltpu.async_copy(g_rows.at[src_v.at[0]], rows0, sem0)
        pltpu.async_copy(g_rows.at[src_v.at[1]], rows1, sem1)

        @pl.loop(0, NCHUNK - 2, step=2)
        def _(j):
            pltpu.make_async_copy(g_rows.at[src_v.at[j]], rows0, sem0).wait()
            pltpu.sync_copy(rows0, acc_rows.at[dst_v.at[j]], add=True)
            pltpu.async_copy(g_rows.at[src_v.at[j + 2]], rows0, sem0)
            pltpu.make_async_copy(g_rows.at[src_v.at[j + 1]], rows1,
                                  sem1).wait()
            pltpu.sync_copy(rows1, acc_rows.at[dst_v.at[j + 1]], add=True)
            pltpu.async_copy(g_rows.at[src_v.at[j + 3]], rows1, sem1)

        pltpu.make_async_copy(g_rows.at[src_v.at[NCHUNK - 2]], rows0,
                              sem0).wait()
        pltpu.sync_copy(rows0, acc_rows.at[dst_v.at[NCHUNK - 2]], add=True)
        pltpu.make_async_copy(g_rows.at[src_v.at[NCHUNK - 1]], rows1,
                              sem1).wait()
        pltpu.sync_copy(rows1, acc_rows.at[dst_v.at[NCHUNK - 1]], add=True)

        plsc.subcore_barrier()
        pltpu.sync_copy(acc_rows.at[pl.ds(s * RPS, RPS)],
                        out_rows.at[c, pl.ds(s * RPS, RPS)])

    return k(g, src3, dst3)


def _blockdiag4(w):
    parts = []
    for kk in range(PK):
        parts.append(jnp.pad(w, ((H * kk, H * (PK - 1 - kk)),
                                 (H * kk, H * (PK - 1 - kk)))))
    return parts[0] + parts[1] + parts[2] + parts[3]


def _tc_xform_body(x_ref, w_ref, o_ref):
    w = w_ref[...]
    blocks = [jnp.dot(x_ref[pl.ds(NP * kk, NP), :], w,
                      preferred_element_type=jnp.float32)
              for kk in range(PK)]
    o_ref[pl.ds(0, NP), :] = jnp.concatenate(blocks, axis=1)
    o_ref[pl.ds(NP, NPP - NP), :] = jnp.zeros((NPP - NP, CH), jnp.float32)


@jax.jit
def _tc_xform(x, w):
    return pl.pallas_call(
        _tc_xform_body,
        out_shape=jax.ShapeDtypeStruct((NPP, CH), jnp.float32),
    )(x, w)


def _tc_scale_body(hist_ref, hp_ref, dis_ref, g_ref):
    deg = hist_ref[0] + hist_ref[1] + 1.0
    dis = lax.rsqrt(deg)
    dis_ref[...] = dis
    g_ref[...] = dis * hp_ref[...]


@jax.jit
def _tc_scale(hist, h1p):
    return pl.pallas_call(
        _tc_scale_body,
        out_shape=[
            jax.ShapeDtypeStruct((NPP, CH), jnp.float32),
            jax.ShapeDtypeStruct((NPP, CH), jnp.float32),
        ],
    )(hist, h1p)


def _tc_layer_body(s_ref, hp_ref, dis_ref, b_ref, w_ref, hp2_ref, g2_ref):
    dis = dis_ref[...]
    ssum = s_ref[0] + s_ref[1]
    out = dis * ssum + dis * dis * hp_ref[...] + b_ref[...]
    h = jnp.maximum(out, 0.0)
    hp2 = jnp.dot(h, _blockdiag4(w_ref[...]),
                  preferred_element_type=jnp.float32)
    hp2_ref[...] = hp2
    g2_ref[...] = dis * hp2


@jax.jit
def _tc_layer(s, hp, dis, b, w):
    return pl.pallas_call(
        _tc_layer_body,
        out_shape=[
            jax.ShapeDtypeStruct((NPP, CH), jnp.float32),
            jax.ShapeDtypeStruct((NPP, CH), jnp.float32),
        ],
    )(s, hp, dis, b, w)


def _tc_head_body(s_ref, hp_ref, dis_ref, b_ref, batch_ref, wl_ref, bl_ref,
                  o_ref):
    dis = dis_ref[pl.ds(0, NP), :]
    ssum = s_ref[0, :NP, :] + s_ref[1, :NP, :]
    out3 = dis * ssum + dis * dis * hp_ref[pl.ds(0, NP), :] + b_ref[...]
    gids = lax.broadcasted_iota(jnp.int32, (NP, G), 1)
    sums = jnp.zeros((G, H), jnp.float32)
    counts = jnp.zeros((G, 1), jnp.float32)
    for kk in range(PK):
        onehot = (batch_ref[:, kk:kk + 1] == gids).astype(jnp.float32)
        sums = sums + lax.dot_general(
            onehot, out3[:, H * kk:H * (kk + 1)],
            (((0,), (0,)), ((), ())), preferred_element_type=jnp.float32)
        counts = counts + jnp.sum(onehot, axis=0)[:, None]
    emb = sums / jnp.maximum(counts, 1.0)
    o_ref[...] = jnp.dot(emb, wl_ref[...],
                         preferred_element_type=jnp.float32) + bl_ref[...]


@jax.jit
def _tc_head(s, hp, dis, b, batch4, wl, bl):
    return pl.pallas_call(
        _tc_head_body,
        out_shape=jax.ShapeDtypeStruct((G, OUT), jnp.float32),
    )(s, hp, dis, b, batch4, wl, bl)


def kernel(x, edge_index, batch, W1, b1, W2, b2, W3, b3, Wl, bl):
    x = x.astype(jnp.float32)
    src = edge_index[0]
    dst = edge_index[1]
    srcp = (src % NP) * PK + src // NP
    dstp = (dst % NP) * PK + dst // NP
    pad = E_PAD - E
    src3 = jnp.concatenate(
        [srcp, jnp.zeros((pad,), src.dtype)]).reshape(NW, NCHUNK, CHE)
    dst3 = jnp.concatenate(
        [dstp, jnp.full((pad,), N, dst.dtype)]).reshape(NW, NCHUNK, CHE)
    batch4 = batch.astype(jnp.int32).reshape(PK, NP).T
    b4 = [jnp.tile(b.reshape(1, H), (1, PK)) for b in (b1, b2, b3)]

    def to_rows(a):
        return a.reshape(NPAD, H)

    def to_packed(sp):
        return sp.reshape(NC, NPP, CH)

    hist = to_packed(_sc_hist(dst3))
    h1p = _tc_xform(x, W1)
    dis, g1 = _tc_scale(hist, h1p)

    s1 = to_packed(_sc_agg(to_rows(g1), src3, dst3))
    h2p, g2 = _tc_layer(s1, h1p, dis, b4[0], W2)
    s2 = to_packed(_sc_agg(to_rows(g2), src3, dst3))
    h3p, g3 = _tc_layer(s2, h2p, dis, b4[1], W3)
    s3 = to_packed(_sc_agg(to_rows(g3), src3, dst3))

    return _tc_head(s3, h3p, dis, b4[2], batch4, Wl, bl.reshape(1, OUT))


# speedup vs baseline: 1.3816x; 1.0743x over previous
import functools

import jax
import jax.numpy as jnp
from jax import lax
from jax.experimental import pallas as pl
from jax.experimental.pallas import tpu as pltpu
from jax.experimental.pallas import tpu_sc as plsc

N = 10000
E = 320000
D_IN = 128
H = 32
OUT = 10
G = 64

NC = 2
NS = 16
NW = NC * NS
CH = 128
CHE = 256
NCHUNK = 2 * (-(-E // (NW * CHE * 2)))
EPW = NCHUNK * CHE
E_PAD = EPW * NW
PK = 4
NP = N // PK
NPAD = 10240
NPP = NPAD // PK
RPS = NPAD // NS
RPP = NPP // NS

_mesh = plsc.VectorSubcoreMesh(core_axis_name="c", subcore_axis_name="s")
_sc_params = pltpu.CompilerParams(use_tc_tiling_on_sc=False)


@jax.jit
def _sc_hist(dst3):
    @functools.partial(
        pl.kernel,
        out_type=jax.ShapeDtypeStruct((NC, NPAD, H), jnp.float32),
        mesh=_mesh,
        scratch_types=[
            pltpu.VMEM((NCHUNK, CHE), jnp.int32),
            pltpu.VMEM((CHE, H), jnp.float32),
            pltpu.VMEM((CH, H), jnp.float32),
            pltpu.VMEM_SHARED((NPAD, H), jnp.float32),
        ],
        compiler_params=_sc_params,
    )
    def k(dst_hbm, out_hbm, dst_v, ones_v, zeros_v, acc_sh):
        c = lax.axis_index("c")
        s = lax.axis_index("s")
        wid = s * NC + c
        acc_rows = acc_sh
        out_rows = out_hbm

        one16 = jnp.ones((16,), jnp.float32)
        zero16 = jnp.zeros((16,), jnp.float32)

        @pl.loop(0, CHE)
        def _(i):
            ones_v[i, pl.ds(0, 16)] = one16
            ones_v[i, pl.ds(16, 16)] = one16

        @pl.loop(0, CH)
        def _(i):
            zeros_v[i, pl.ds(0, 16)] = zero16
            zeros_v[i, pl.ds(16, 16)] = zero16

        @pl.loop(0, RPS // CH)
        def _(t):
            pltpu.sync_copy(zeros_v, acc_rows.at[pl.ds(s * RPS + t * CH, CH)])

        plsc.subcore_barrier()

        pltpu.sync_copy(dst_hbm.at[wid], dst_v)

        @pl.loop(0, NCHUNK)
        def _(j):
            pltpu.sync_copy(ones_v, acc_rows.at[dst_v.at[j]], add=True)

        plsc.subcore_barrier()
        pltpu.sync_copy(acc_rows.at[pl.ds(s * RPS, RPS)],
                        out_rows.at[c, pl.ds(s * RPS, RPS)])

    return k(dst3)


@jax.jit
def _sc_agg(g, src3, dst3):
    @functools.partial(
        pl.kernel,
        out_type=jax.ShapeDtypeStruct((NC, NPAD, H), jnp.float32),
        mesh=_mesh,
        scratch_types=[
            pltpu.VMEM((NCHUNK, CHE), jnp.int32),
            pltpu.VMEM((NCHUNK, CHE), jnp.int32),
            pltpu.VMEM((CHE, H), jnp.float32),
            pltpu.VMEM((CHE, H), jnp.float32),
            pltpu.SemaphoreType.DMA,
            pltpu.SemaphoreType.DMA,
            pltpu.VMEM_SHARED((NPAD, H), jnp.float32),
            pltpu.VMEM_SHARED((NPAD, H), jnp.float32),
        ],
        compiler_params=_sc_params,
    )
    def k(g_hbm, src_hbm, dst_hbm, out_hbm, src_v, dst_v, rows0, rows1,
          sem0, sem1, acc_sh, g_sh):
        c = lax.axis_index("c")
        s = lax.axis_index("s")
        wid = s * NC + c
        acc_rows = acc_sh
        g_rows = g_sh
        g_hbm_rows = g_hbm
        out_rows = out_hbm

        zero16 = jnp.zeros((16,), jnp.float32)

        @pl.loop(0, CH)
        def _(i):
            rows0[i, pl.ds(0, 16)] = zero16
            rows0[i, pl.ds(16, 16)] = zero16

        @pl.loop(0, RPS // CH)
        def _(t):
            pltpu.sync_copy(rows0.at[pl.ds(0, CH)],
                            acc_rows.at[pl.ds(s * RPS + t * CH, CH)])

        pltpu.sync_copy(g_hbm_rows.at[pl.ds(s * RPS, RPS)],
                        g_sh.at[pl.ds(s * RPS, RPS)])

        plsc.subcore_barrier()

        pltpu.sync_copy(src_hbm.at[wid], src_v)
        pltpu.sync_copy(dst_hbm.at[wid], dst_v)

        pltpu.async_copy(g_rows.at[src_v.at[0]], rows0, sem0)
        pltpu.async_copy(g_rows.at[src_v.at[1]], rows1, sem1)

        @pl.loop(0, NCHUNK - 2, step=2)
        def _(j):
            pltpu.make_async_copy(g_rows.at[src_v.at[j]], rows0, sem0).wait()
            pltpu.sync_copy(rows0, acc_rows.at[dst_v.at[j]], add=True)
            pltpu.async_copy(g_rows.at[src_v.at[j + 2]], rows0, sem0)
            pltpu.make_async_copy(g_rows.at[src_v.at[j + 1]], rows1,
                                  sem1).wait()
            pltpu.sync_copy(rows1, acc_rows.at[dst_v.at[j + 1]], add=True)
            pltpu.async_copy(g_rows.at[src_v.at[j + 3]], rows1, sem1)

        pltpu.make_async_copy(g_rows.at[src_v.at[NCHUNK - 2]], rows0,
                              sem0).wait()
        pltpu.sync_copy(rows0, acc_rows.at[dst_v.at[NCHUNK - 2]], add=True)
        pltpu.make_async_copy(g_rows.at[src_v.at[NCHUNK - 1]], rows1,
                              sem1).wait()
        pltpu.sync_copy(rows1, acc_rows.at[dst_v.at[NCHUNK - 1]], add=True)

        plsc.subcore_barrier()
        pltpu.sync_copy(acc_rows.at[pl.ds(s * RPS, RPS)],
                        out_rows.at[c, pl.ds(s * RPS, RPS)])

    return k(g, src3, dst3)


def _blockdiag4(w):
    parts = []
    for kk in range(PK):
        parts.append(jnp.pad(w, ((H * kk, H * (PK - 1 - kk)),
                                 (H * kk, H * (PK - 1 - kk)))))
    return parts[0] + parts[1] + parts[2] + parts[3]


EROWS = E // CHE
ROWS_PAD = E_PAD // CHE
MAGIC = 13422


def _tc_prep_body(src_ref, dst_ref, os_ref, od_ref):
    def perm(v):
        q = lax.shift_right_logical(v * MAGIC, 25)
        return v * PK - (N - 1) * q

    os_ref[pl.ds(0, EROWS), :] = perm(src_ref[...])
    od_ref[pl.ds(0, EROWS), :] = perm(dst_ref[...])
    os_ref[pl.ds(EROWS, ROWS_PAD - EROWS), :] = jnp.zeros(
        (ROWS_PAD - EROWS, CHE), jnp.int32)
    od_ref[pl.ds(EROWS, ROWS_PAD - EROWS), :] = jnp.full(
        (ROWS_PAD - EROWS, CHE), N, jnp.int32)


@jax.jit
def _tc_prep(src2d, dst2d):
    return pl.pallas_call(
        _tc_prep_body,
        out_shape=[
            jax.ShapeDtypeStruct((ROWS_PAD, CHE), jnp.int32),
            jax.ShapeDtypeStruct((ROWS_PAD, CHE), jnp.int32),
        ],
    )(src2d, dst2d)


def _tc_xform_body(x_ref, w_ref, o_ref):
    w = w_ref[...]
    blocks = [jnp.dot(x_ref[pl.ds(NP * kk, NP), :], w,
                      preferred_element_type=jnp.float32)
              for kk in range(PK)]
    o_ref[pl.ds(0, NP), :] = jnp.concatenate(blocks, axis=1)
    o_ref[pl.ds(NP, NPP - NP), :] = jnp.zeros((NPP - NP, CH), jnp.float32)


@jax.jit
def _tc_xform(x, w):
    return pl.pallas_call(
        _tc_xform_body,
        out_shape=jax.ShapeDtypeStruct((NPP, CH), jnp.float32),
    )(x, w)


def _tc_scale_body(hist_ref, hp_ref, dis_ref, g_ref):
    deg = hist_ref[0] + hist_ref[1] + 1.0
    dis = lax.rsqrt(deg)
    dis_ref[...] = dis
    g_ref[...] = dis * hp_ref[...]


@jax.jit
def _tc_scale(hist, h1p):
    return pl.pallas_call(
        _tc_scale_body,
        out_shape=[
            jax.ShapeDtypeStruct((NPP, CH), jnp.float32),
            jax.ShapeDtypeStruct((NPP, CH), jnp.float32),
        ],
    )(hist, h1p)


def _tc_layer_body(s_ref, hp_ref, dis_ref, b_ref, w_ref, hp2_ref, g2_ref):
    dis = dis_ref[...]
    ssum = s_ref[0] + s_ref[1]
    out = dis * ssum + dis * dis * hp_ref[...] + b_ref[...]
    h = jnp.maximum(out, 0.0)
    hp2 = jnp.dot(h, _blockdiag4(w_ref[...]),
                  preferred_element_type=jnp.float32)
    hp2_ref[...] = hp2
    g2_ref[...] = dis * hp2


@jax.jit
def _tc_layer(s, hp, dis, b, w):
    return pl.pallas_call(
        _tc_layer_body,
        out_shape=[
            jax.ShapeDtypeStruct((NPP, CH), jnp.float32),
            jax.ShapeDtypeStruct((NPP, CH), jnp.float32),
        ],
    )(s, hp, dis, b, w)


def _tc_head_body(s_ref, hp_ref, dis_ref, b_ref, batch_ref, wl_ref, bl_ref,
                  o_ref):
    dis = dis_ref[pl.ds(0, NP), :]
    ssum = s_ref[0, :NP, :] + s_ref[1, :NP, :]
    out3 = dis * ssum + dis * dis * hp_ref[pl.ds(0, NP), :] + b_ref[...]
    gids = lax.broadcasted_iota(jnp.int32, (G, NP), 0)
    sums = jnp.zeros((G, H), jnp.float32)
    counts = jnp.zeros((G, 1), jnp.float32)
    for kk in range(PK):
        onehot = (batch_ref[kk:kk + 1, :] == gids).astype(jnp.float32)
        sums = sums + jnp.dot(onehot, out3[:, H * kk:H * (kk + 1)],
                              preferred_element_type=jnp.float32)
        counts = counts + jnp.sum(onehot, axis=1)[:, None]
    emb = sums / jnp.maximum(counts, 1.0)
    o_ref[...] = jnp.dot(emb, wl_ref[...],
                         preferred_element_type=jnp.float32) + bl_ref[...]


@jax.jit
def _tc_head(s, hp, dis, b, batch4, wl, bl):
    return pl.pallas_call(
        _tc_head_body,
        out_shape=jax.ShapeDtypeStruct((G, OUT), jnp.float32),
    )(s, hp, dis, b, batch4, wl, bl)


def kernel(x, edge_index, batch, W1, b1, W2, b2, W3, b3, Wl, bl):
    x = x.astype(jnp.float32)
    srcp2, dstp2 = _tc_prep(edge_index[0].reshape(EROWS, CHE),
                            edge_index[1].reshape(EROWS, CHE))
    src3 = srcp2.reshape(NW, NCHUNK, CHE)
    dst3 = dstp2.reshape(NW, NCHUNK, CHE)
    batch2 = batch.astype(jnp.int32).reshape(PK, NP)
    b4 = [jnp.tile(b.reshape(1, H), (1, PK)) for b in (b1, b2, b3)]

    def to_rows(a):
        return a.reshape(NPAD, H)

    def to_packed(sp):
        return sp.reshape(NC, NPP, CH)

    hist = to_packed(_sc_hist(dst3))
    h1p = _tc_xform(x, W1)
    dis, g1 = _tc_scale(hist, h1p)

    s1 = to_packed(_sc_agg(to_rows(g1), src3, dst3))
    h2p, g2 = _tc_layer(s1, h1p, dis, b4[0], W2)
    s2 = to_packed(_sc_agg(to_rows(g2), src3, dst3))
    h3p, g3 = _tc_layer(s2, h2p, dis, b4[1], W3)
    s3 = to_packed(_sc_agg(to_rows(g3), src3, dst3))

    return _tc_head(s3, h3p, dis, b4[2], batch2, Wl, bl.reshape(1, OUT))


# speedup vs baseline: 1.4815x; 1.0723x over previous
import functools

import jax
import jax.numpy as jnp
from jax import lax
from jax.experimental import pallas as pl
from jax.experimental.pallas import tpu as pltpu
from jax.experimental.pallas import tpu_sc as plsc

N = 10000
E = 320000
D_IN = 128
H = 32
OUT = 10
G = 64

NC = 2
NS = 16
NW = NC * NS
CH = 128
CHE = 256
NCHUNK = 2 * (-(-E // (NW * CHE * 2)))
EPW = NCHUNK * CHE
E_PAD = EPW * NW
PK = 4
NP = N // PK
NPAD = 10240
NPP = NPAD // PK
RPS = NPAD // NS
RPP = NPP // NS

_mesh = plsc.VectorSubcoreMesh(core_axis_name="c", subcore_axis_name="s")
_sc_params = pltpu.CompilerParams(use_tc_tiling_on_sc=False)


@jax.jit
def _sc_hist(dst3):
    @functools.partial(
        pl.kernel,
        out_type=jax.ShapeDtypeStruct((NC, NPAD, H), jnp.float32),
        mesh=_mesh,
        scratch_types=[
            pltpu.VMEM((NCHUNK, CHE), jnp.int32),
            pltpu.VMEM((CHE, H), jnp.float32),
            pltpu.VMEM((CH, H), jnp.float32),
            pltpu.VMEM_SHARED((NPAD, H), jnp.float32),
        ],
        compiler_params=_sc_params,
    )
    def k(dst_hbm, out_hbm, dst_v, ones_v, zeros_v, acc_sh):
        c = lax.axis_index("c")
        s = lax.axis_index("s")
        wid = s * NC + c
        acc_rows = acc_sh
        out_rows = out_hbm

        one16 = jnp.ones((16,), jnp.float32)
        zero16 = jnp.zeros((16,), jnp.float32)

        @pl.loop(0, CHE)
        def _(i):
            ones_v[i, pl.ds(0, 16)] = one16
            ones_v[i, pl.ds(16, 16)] = one16

        @pl.loop(0, CH)
        def _(i):
            zeros_v[i, pl.ds(0, 16)] = zero16
            zeros_v[i, pl.ds(16, 16)] = zero16

        @pl.loop(0, RPS // CH)
        def _(t):
            pltpu.sync_copy(zeros_v, acc_rows.at[pl.ds(s * RPS + t * CH, CH)])

        plsc.subcore_barrier()

        pltpu.sync_copy(dst_hbm.at[wid], dst_v)

        @pl.loop(0, NCHUNK)
        def _(j):
            pltpu.sync_copy(ones_v, acc_rows.at[dst_v.at[j]], add=True)

        plsc.subcore_barrier()
        pltpu.sync_copy(acc_rows.at[pl.ds(s * RPS, RPS)],
                        out_rows.at[c, pl.ds(s * RPS, RPS)])

    return k(dst3)


@jax.jit
def _sc_agg(g, src3, dst3):
    @functools.partial(
        pl.kernel,
        out_type=jax.ShapeDtypeStruct((NC, NPAD, H), jnp.float32),
        mesh=_mesh,
        scratch_types=[
            pltpu.VMEM((NCHUNK, CHE), jnp.int32),
            pltpu.VMEM((NCHUNK, CHE), jnp.int32),
            pltpu.VMEM((CHE, H), jnp.float32),
            pltpu.VMEM((CHE, H), jnp.float32),
            pltpu.SemaphoreType.DMA,
            pltpu.SemaphoreType.DMA,
            pltpu.VMEM_SHARED((NPAD, H), jnp.float32),
            pltpu.VMEM_SHARED((NPAD, H), jnp.float32),
        ],
        compiler_params=_sc_params,
    )
    def k(g_hbm, src_hbm, dst_hbm, out_hbm, src_v, dst_v, rows0, rows1,
          sem0, sem1, acc_sh, g_sh):
        c = lax.axis_index("c")
        s = lax.axis_index("s")
        wid = s * NC + c
        acc_rows = acc_sh
        g_rows = g_sh
        g_hbm_rows = g_hbm
        out_rows = out_hbm

        zero16 = jnp.zeros((16,), jnp.float32)

        @pl.loop(0, CH)
        def _(i):
            rows0[i, pl.ds(0, 16)] = zero16
            rows0[i, pl.ds(16, 16)] = zero16

        @pl.loop(0, RPS // CH)
        def _(t):
            pltpu.sync_copy(rows0.at[pl.ds(0, CH)],
                            acc_rows.at[pl.ds(s * RPS + t * CH, CH)])

        pltpu.sync_copy(g_hbm_rows.at[pl.ds(s * RPS, RPS)],
                        g_sh.at[pl.ds(s * RPS, RPS)])

        plsc.subcore_barrier()

        pltpu.sync_copy(src_hbm.at[wid], src_v)
        pltpu.sync_copy(dst_hbm.at[wid], dst_v)

        pltpu.async_copy(g_rows.at[src_v.at[0]], rows0, sem0)
        pltpu.async_copy(g_rows.at[src_v.at[1]], rows1, sem1)

        @pl.loop(0, NCHUNK - 2, step=2)
        def _(j):
            pltpu.make_async_copy(g_rows.at[src_v.at[j]], rows0, sem0).wait()
            pltpu.sync_copy(rows0, acc_rows.at[dst_v.at[j]], add=True)
            pltpu.async_copy(g_rows.at[src_v.at[j + 2]], rows0, sem0)
            pltpu.make_async_copy(g_rows.at[src_v.at[j + 1]], rows1,
                                  sem1).wait()
            pltpu.sync_copy(rows1, acc_rows.at[dst_v.at[j + 1]], add=True)
            pltpu.async_copy(g_rows.at[src_v.at[j + 3]], rows1, sem1)

        pltpu.make_async_copy(g_rows.at[src_v.at[NCHUNK - 2]], rows0,
                              sem0).wait()
        pltpu.sync_copy(rows0, acc_rows.at[dst_v.at[NCHUNK - 2]], add=True)
        pltpu.make_async_copy(g_rows.at[src_v.at[NCHUNK - 1]], rows1,
                              sem1).wait()
        pltpu.sync_copy(rows1, acc_rows.at[dst_v.at[NCHUNK - 1]], add=True)

        plsc.subcore_barrier()
        pltpu.sync_copy(acc_rows.at[pl.ds(s * RPS, RPS)],
                        out_rows.at[c, pl.ds(s * RPS, RPS)])

    return k(g, src3, dst3)


def _blockdiag4(w):
    parts = []
    for kk in range(PK):
        parts.append(jnp.pad(w, ((H * kk, H * (PK - 1 - kk)),
                                 (H * kk, H * (PK - 1 - kk)))))
    return parts[0] + parts[1] + parts[2] + parts[3]


EROWS = E // CH
ROWS_PAD = E_PAD // CH
MAGIC = 13422


def _tc_prep_body(src_ref, dst_ref, os_ref, od_ref):
    def perm(v):
        q = lax.shift_right_logical(v * MAGIC, 25)
        return v * PK - (N - 1) * q

    os_ref[pl.ds(0, EROWS), :] = perm(
        src_ref[...].reshape(EROWS, CH))
    od_ref[pl.ds(0, EROWS), :] = perm(
        dst_ref[...].reshape(EROWS, CH))
    os_ref[pl.ds(EROWS, ROWS_PAD - EROWS), :] = jnp.zeros(
        (ROWS_PAD - EROWS, CH), jnp.int32)
    od_ref[pl.ds(EROWS, ROWS_PAD - EROWS), :] = jnp.full(
        (ROWS_PAD - EROWS, CH), N, jnp.int32)


@jax.jit
def _tc_prep(ei3):
    return pl.pallas_call(
        _tc_prep_body,
        grid=(1,),
        in_specs=[
            pl.BlockSpec((1, 1, E), lambda i: (0, 0, 0)),
            pl.BlockSpec((1, 1, E), lambda i: (1, 0, 0)),
        ],
        out_specs=[
            pl.BlockSpec((ROWS_PAD, CH), lambda i: (0, 0)),
            pl.BlockSpec((ROWS_PAD, CH), lambda i: (0, 0)),
        ],
        out_shape=[
            jax.ShapeDtypeStruct((ROWS_PAD, CH), jnp.int32),
            jax.ShapeDtypeStruct((ROWS_PAD, CH), jnp.int32),
        ],
    )(ei3, ei3)


def _tc_xform_body(x_ref, w_ref, o_ref):
    w = w_ref[...]
    blocks = [jnp.dot(x_ref[pl.ds(NP * kk, NP), :], w,
                      preferred_element_type=jnp.float32)
              for kk in range(PK)]
    o_ref[pl.ds(0, NP), :] = jnp.concatenate(blocks, axis=1)
    o_ref[pl.ds(NP, NPP - NP), :] = jnp.zeros((NPP - NP, CH), jnp.float32)


@jax.jit
def _tc_xform(x, w):
    return pl.pallas_call(
        _tc_xform_body,
        out_shape=jax.ShapeDtypeStruct((NPP, CH), jnp.float32),
    )(x, w)


def _tc_scale_body(hist_ref, hp_ref, dis_ref, g_ref):
    deg = hist_ref[0] + hist_ref[1] + 1.0
    dis = lax.rsqrt(deg)
    dis_ref[...] = dis
    g_ref[...] = dis * hp_ref[...]


@jax.jit
def _tc_scale(hist, h1p):
    return pl.pallas_call(
        _tc_scale_body,
        out_shape=[
            jax.ShapeDtypeStruct((NPP, CH), jnp.float32),
            jax.ShapeDtypeStruct((NPP, CH), jnp.float32),
        ],
    )(hist, h1p)


def _tc_layer_body(s_ref, hp_ref, dis_ref, b_ref, w_ref, hp2_ref, g2_ref):
    dis = dis_ref[...]
    ssum = s_ref[0] + s_ref[1]
    out = dis * ssum + dis * dis * hp_ref[...] + b_ref[...]
    h = jnp.maximum(out, 0.0)
    hp2 = jnp.dot(h, _blockdiag4(w_ref[...]),
                  preferred_element_type=jnp.float32)
    hp2_ref[...] = hp2
    g2_ref[...] = dis * hp2


@jax.jit
def _tc_layer(s, hp, dis, b, w):
    return pl.pallas_call(
        _tc_layer_body,
        out_shape=[
            jax.ShapeDtypeStruct((NPP, CH), jnp.float32),
            jax.ShapeDtypeStruct((NPP, CH), jnp.float32),
        ],
    )(s, hp, dis, b, w)


def _tc_head_body(s_ref, hp_ref, dis_ref, b_ref, batch_ref, wl_ref, bl_ref,
                  o_ref):
    dis = dis_ref[pl.ds(0, NP), :]
    ssum = s_ref[0, :NP, :] + s_ref[1, :NP, :]
    out3 = dis * ssum + dis * dis * hp_ref[pl.ds(0, NP), :] + b_ref[...]
    gids = lax.broadcasted_iota(jnp.int32, (G, NP), 0)
    sums = jnp.zeros((G, H), jnp.float32)
    counts = jnp.zeros((G, 1), jnp.float32)
    for kk in range(PK):
        onehot = (batch_ref[kk:kk + 1, :] == gids).astype(jnp.float32)
        sums = sums + jnp.dot(onehot, out3[:, H * kk:H * (kk + 1)],
                              preferred_element_type=jnp.float32)
        counts = counts + jnp.sum(onehot, axis=1)[:, None]
    emb = sums / jnp.maximum(counts, 1.0)
    o_ref[...] = jnp.dot(emb, wl_ref[...],
                         preferred_element_type=jnp.float32) + bl_ref[...]


@jax.jit
def _tc_head(s, hp, dis, b, batch4, wl, bl):
    return pl.pallas_call(
        _tc_head_body,
        out_shape=jax.ShapeDtypeStruct((G, OUT), jnp.float32),
    )(s, hp, dis, b, batch4, wl, bl)


def kernel(x, edge_index, batch, W1, b1, W2, b2, W3, b3, Wl, bl):
    x = x.astype(jnp.float32)
    srcp2, dstp2 = _tc_prep(edge_index.reshape(2, 1, E))
    src3 = srcp2.reshape(NW, NCHUNK, CHE)
    dst3 = dstp2.reshape(NW, NCHUNK, CHE)
    batch2 = batch.astype(jnp.int32).reshape(PK, NP)
    b4 = [jnp.tile(b.reshape(1, H), (1, PK)) for b in (b1, b2, b3)]

    def to_rows(a):
        return a.reshape(NPAD, H)

    def to_packed(sp):
        return sp.reshape(NC, NPP, CH)

    hist = to_packed(_sc_hist(dst3))
    h1p = _tc_xform(x, W1)
    dis, g1 = _tc_scale(hist, h1p)

    s1 = to_packed(_sc_agg(to_rows(g1), src3, dst3))
    h2p, g2 = _tc_layer(s1, h1p, dis, b4[0], W2)
    s2 = to_packed(_sc_agg(to_rows(g2), src3, dst3))
    h3p, g3 = _tc_layer(s2, h2p, dis, b4[1], W3)
    s3 = to_packed(_sc_agg(to_rows(g3), src3, dst3))

    return _tc_head(s3, h3p, dis, b4[2], batch2, Wl, bl.reshape(1, OUT))
